# baseline (speedup 1.0000x reference)
import jax
import jax.numpy as jnp
from jax import lax
from jax.experimental import pallas as pl
from jax.experimental.pallas import tpu as pltpu
from jax.experimental.pallas import tpu_sc as plsc

N_POINTS = 1048576
NCORES = 2
NSUB = 16
NW = NCORES * NSUB
PW = N_POINTS // NW
P = 128
NB = PW // P
L = 16

RES = (128, 256, 512, 1024)
GRID_DEFS = tuple((a, b, R) for (a, b) in ((0, 1), (0, 2), (1, 2)) for R in RES)
NG = len(GRID_DEFS)
NF = 2
NOUT = NG * NF


def _sc_body(xt, *rest):
    tables = rest[:NG]
    out = rest[NG]
    coords, idxb, wgt, dstb, acc, sem = rest[NG + 1:]

    wid = lax.axis_index("s") * NCORES + lax.axis_index("c")
    iot = lax.iota(jnp.int32, L)
    pat_pt = lax.shift_right_logical(iot, 1)
    pat_f = lax.bitwise_and(iot, 1)

    def batch_body(b, carry):
        base = wid * PW + b * P
        pltpu.sync_copy(xt.at[:, pl.ds(base, P)], coords)

        descs = []
        for g, (ca, cb, R) in enumerate(GRID_DEFS):
            @plsc.parallel_loop(0, P // L, 1, unroll=2)
            def idx_body(j, g=g, ca=ca, cb=cb, R=R):
                sl = pl.ds(j * L, L)
                pu = coords[ca, sl] * jnp.float32(R - 1)
                pv = coords[cb, sl] * jnp.float32(R - 1)
                r0 = pu.astype(jnp.int32)
                c0 = pv.astype(jnp.int32)
                idxb[g, sl] = r0 * R + c0
                wgt[2 * g + 0, sl] = pu - r0.astype(jnp.float32)
                wgt[2 * g + 1, sl] = pv - c0.astype(jnp.float32)

            descs.append(pltpu.async_copy(
                tables[g].at[idxb.at[g]], dstb.at[g], sem))
        for d in descs:
            d.wait()

        for g in range(NG):
            gsp = jnp.full((L,), g, jnp.int32)
            cp0 = pat_f
            cp1 = pat_f + 2
            cp2 = pat_f + 4
            cp3 = pat_f + 6
            cpat = pat_f + 2 * g
            wr = jnp.full((L,), 2 * g + 0, jnp.int32)
            wc = jnp.full((L,), 2 * g + 1, jnp.int32)

            @plsc.parallel_loop(0, (P * NF) // L, 1, unroll=4)
            def interp_body(j, g=g, gsp=gsp, cpat=cpat, wr=wr, wc=wc,
                            cp0=cp0, cp1=cp1, cp2=cp2, cp3=cp3):
                pt = pat_pt + j * (L // 2)
                g00 = plsc.load_gather(dstb, [gsp, pt, cp0])
                g01 = plsc.load_gather(dstb, [gsp, pt, cp1])
                g10 = plsc.load_gather(dstb, [gsp, pt, cp2])
                g11 = plsc.load_gather(dstb, [gsp, pt, cp3])
                fr = plsc.load_gather(wgt, [wr, pt])
                fc = plsc.load_gather(wgt, [wc, pt])
                h0 = g00 + fc * (g01 - g00)
                h1 = g10 + fc * (g11 - g10)
                res = h0 + fr * (h1 - h0)
                plsc.store_scatter(acc, [pt, cpat], res)

        pltpu.sync_copy(acc, out.at[pl.ds(base, P)])
        return carry

    lax.fori_loop(0, NB, batch_body, 0)


def _pack_t8(g, R):
    rr = R * R
    t = g.reshape(rr, NF)
    tp = jnp.concatenate([t, jnp.zeros((R + 1, NF), jnp.float32)], axis=0)
    return jnp.concatenate(
        [tp[:rr], tp[1:rr + 1], tp[R:rr + R], tp[R + 1:rr + R + 1]], axis=1)


def kernel(x, bound, xy_g0, xy_g1, xy_g2, xy_g3,
           yz_g0, yz_g1, yz_g2, yz_g3,
           xz_g0, xz_g1, xz_g2, xz_g3):
    del bound
    xt = x.T
    grids = (xy_g0, xy_g1, xy_g2, xy_g3,
             yz_g0, yz_g1, yz_g2, yz_g3,
             xz_g0, xz_g1, xz_g2, xz_g3)
    tabs = [_pack_t8(g, R) for g, (_, _, R) in zip(grids, GRID_DEFS)]

    f = pl.kernel(
        _sc_body,
        out_type=jax.ShapeDtypeStruct((N_POINTS, NOUT), jnp.float32),
        mesh=plsc.VectorSubcoreMesh(
            core_axis_name="c", subcore_axis_name="s",
            num_cores=NCORES, num_subcores=NSUB),
        scratch_types=[
            pltpu.VMEM((3, P), jnp.float32),
            pltpu.VMEM((NG, P), jnp.int32),
            pltpu.VMEM((2 * NG, P), jnp.float32),
            pltpu.VMEM((NG, P, 8), jnp.float32),
            pltpu.VMEM((P, NOUT), jnp.float32),
            pltpu.SemaphoreType.DMA,
        ],
        compiler_params=pltpu.CompilerParams(
            needs_layout_passes=False, use_tc_tiling_on_sc=False),
    )
    return f(xt, *tabs)
